# baseline (speedup 1.0000x reference)
import jax
import jax.numpy as jnp
from jax import lax
from jax.experimental import pallas as pl
from jax.experimental.pallas import tpu as pltpu
from jax.experimental.pallas import tpu_sc as plsc

E = 64
EC = E // 16
NPOS = 200
NC = 2
NS = 16
NW = NC * NS
SEQ_PER_CHUNK = 2
CHUNK = SEQ_PER_CHUNK * NPOS
SUB = 128
GROUP = 16
RW = 16384
RH = RW // 2


def _perm(v, idx):
    return v.at[idx].get(mode="promise_in_bounds")


def _repack_body(x_ref, o_ref):
    x = x_ref[...]
    hw = x.shape[1] // 2
    o_ref[:, 0:E] = x[:, 0:hw].T
    o_ref[:, E:2 * E] = x[:, hw:2 * hw].T


def _repack(tab_t):
    v = tab_t.shape[1]
    grid = (v + RW - 1) // RW
    return pl.pallas_call(
        _repack_body,
        grid=(grid,),
        in_specs=[pl.BlockSpec((E, RW), lambda i: (0, i))],
        out_specs=pl.BlockSpec((RH, 2 * E), lambda i: (i, 0)),
        out_shape=jax.ShapeDtypeStruct((grid * RH, 2 * E), jnp.float32),
    )(tab_t)


def _sc_embed_ln(seq1d, lbl, tab2, seg_table, pe, a2, b2, n_b, n_l):
    n_rows = n_b * n_l
    rows_per_w = n_rows // NW
    seqs_per_w = rows_per_w // NPOS
    n_chunks = rows_per_w // CHUNK
    sub_sizes = []
    off = 0
    while off < CHUNK:
        sub_sizes.append(min(SUB, CHUNK - off))
        off += min(SUB, CHUNK - off)
    mesh = plsc.VectorSubcoreMesh(core_axis_name="c", subcore_axis_name="s")

    def body(seq_hbm, lbl_hbm, tab_hbm, seg_hbm, pe_hbm, a2_hbm, b2_hbm,
             out_hbm, idx_s, idx_p, lblv, xbuf, ybuf, pebuf, segbuf,
             a2v_m, b2v_m, sem):
        wid = lax.axis_index("s") * NC + lax.axis_index("c")
        pltpu.sync_copy(pe_hbm.at[0, pl.ds(0, NPOS)], pebuf)
        pltpu.sync_copy(seg_hbm, segbuf)
        pltpu.sync_copy(a2_hbm, a2v_m)
        pltpu.sync_copy(b2_hbm, b2v_m)
        a2v = [a2v_m[pl.ds(c * 16, 16)] for c in range(EC)]
        b2v = [b2v_m[pl.ds(c * 16, 16)] for c in range(EC)]
        seg0 = [segbuf[0, pl.ds(c * 16, 16)] for c in range(EC)]
        dseg = [segbuf[1, pl.ds(c * 16, 16)] - seg0[c] for c in range(EC)]
        iota = jnp.arange(16, dtype=jnp.int32)
        bfly = [iota ^ m for m in (8, 4, 2, 1)]
        lanes = [jnp.full((16,), u, dtype=jnp.int32) for u in range(GROUP)]

        def hsum(v):
            for bidx in bfly:
                v = v + _perm(v, bidx)
            return v

        def process_row(r, svf, h):
            sq = jnp.where(r >= NPOS, 1, 0).astype(jnp.int32)
            l = r - sq * NPOS
            xp = []
            for c in range(EC):
                x = xbuf[r, pl.ds(h * E + c * 16, 16)]
                p = pebuf[l, pl.ds(c * 16, 16)]
                sg = svf * dseg[c] + seg0[c]
                xp.append((x + p) + sg)
            s1v = (xp[0] + xp[1]) + (xp[2] + xp[3])
            s2v = (xp[0] * xp[0] + xp[1] * xp[1]) + (xp[2] * xp[2] + xp[3] * xp[3])
            st = hsum(s1v)
            sst = hsum(s2v)
            mean = st * (1.0 / E)
            var = (sst - st * mean) * (1.0 / (E - 1))
            v = jnp.maximum(var, 1e-20)
            i = lax.bitcast_convert_type(v, jnp.int32)
            i = jnp.int32(0x5F3759DF) - (i >> 1)
            rsq = lax.bitcast_convert_type(i, jnp.float32)
            rsq = rsq * (1.5 - 0.5 * v * rsq * rsq)
            d = v * rsq + 1e-6
            inv = rsq * (2.0 - d * rsq)
            for c in range(EC):
                y = (xp[c] - mean) * inv * a2v[c] + b2v[c]
                ybuf[sq, l, pl.ds(c * 16, 16)] = y

        def chunk_body(ch, carry):
            base = wid * rows_per_w + ch * CHUNK
            b0 = wid * seqs_per_w + ch * SEQ_PER_CHUNK
            pltpu.sync_copy(seq_hbm.at[pl.ds(base, CHUNK)], idx_s)
            pltpu.sync_copy(lbl_hbm.at[pl.ds(base, CHUNK)], lblv)

            @plsc.parallel_loop(0, CHUNK // 16)
            def pidx_body(k):
                ids = idx_s[pl.ds(k * 16, 16)]
                idx_p[pl.ds(k * 16, 16)] = (
                    ((ids >> 14) << 13) | (ids & (RH - 1)))
            copies = []
            off = 0
            for sz in sub_sizes:
                copies.append(pltpu.async_copy(
                    tab_hbm.at[idx_p.at[pl.ds(off, sz)]],
                    xbuf.at[pl.ds(off, sz)], sem))
                off += sz
            for cp in copies:
                cp.wait()

            @plsc.parallel_loop(0, CHUNK // GROUP, 1, unroll=4)
            def row_body(g):
                lblf = lblv[pl.ds(g * GROUP, GROUP)].astype(jnp.float32)
                hvec = (idx_s[pl.ds(g * GROUP, GROUP)] >> 13) & 1
                for u in range(GROUP):
                    process_row(g * GROUP + u, _perm(lblf, lanes[u]), hvec[u])
            pltpu.sync_copy(ybuf, out_hbm.at[pl.ds(b0, SEQ_PER_CHUNK)])
            return carry

        lax.fori_loop(0, n_chunks, chunk_body, 0)

    call = pl.kernel(
        body,
        out_type=jax.ShapeDtypeStruct((n_b, n_l, E), jnp.float32),
        mesh=mesh,
        scratch_types=[
            pltpu.VMEM((CHUNK,), jnp.int32),
            pltpu.VMEM((CHUNK,), jnp.int32),
            pltpu.VMEM((CHUNK,), jnp.int32),
            pltpu.VMEM((CHUNK, 2 * E), jnp.float32),
            pltpu.VMEM((SEQ_PER_CHUNK, NPOS, E), jnp.float32),
            pltpu.VMEM((NPOS, E), jnp.float32),
            pltpu.VMEM((2, E), jnp.float32),
            pltpu.VMEM((E,), jnp.float32),
            pltpu.VMEM((E,), jnp.float32),
            pltpu.SemaphoreType.DMA,
        ],
        compiler_params=pltpu.CompilerParams(use_tc_tiling_on_sc=True),
    )
    return call(seq1d, lbl, tab2, seg_table, pe, a2, b2)


def kernel(sequence, segment_label, token_table, segment_table, pe, a_2, b_2):
    b, l = sequence.shape
    n_rows = b * l
    seq1d = sequence.reshape(n_rows)
    lbl = segment_label.reshape(n_rows)
    tab2 = _repack(token_table.T)
    return _sc_embed_ln(seq1d, lbl, tab2, segment_table, pe,
                        a_2, b_2, b, l)


# speedup vs baseline: 1.3339x; 1.3339x over previous
import jax
import jax.numpy as jnp
from jax import lax
from jax.experimental import pallas as pl
from jax.experimental.pallas import tpu as pltpu
from jax.experimental.pallas import tpu_sc as plsc

E = 64
EC = E // 16
NPOS = 200
NC = 2
NS = 16
NW = NC * NS
SEQ_PER_CHUNK = 2
CHUNK = SEQ_PER_CHUNK * NPOS
SUB = 128
GROUP = 16
RW = 8192
RH = RW // 2


def _perm(v, idx):
    return v.at[idx].get(mode="promise_in_bounds")


def _repack_body(x_ref, o_ref):
    x = x_ref[...]
    hw = x.shape[1] // 2
    o_ref[:, 0:E] = x[:, 0:hw].T
    o_ref[:, E:2 * E] = x[:, hw:2 * hw].T


def _repack(tab_t):
    v = tab_t.shape[1]
    grid = (v + RW - 1) // RW
    return pl.pallas_call(
        _repack_body,
        grid=(grid,),
        in_specs=[pl.BlockSpec((E, RW), lambda i: (0, i))],
        out_specs=pl.BlockSpec((RH, 2 * E), lambda i: (i, 0)),
        out_shape=jax.ShapeDtypeStruct((grid * RH, 2 * E), jnp.float32),
    )(tab_t)


def _sc_embed_ln(seq1d, lbl, tab2, seg_table, pe, a2, b2, n_b, n_l):
    n_rows = n_b * n_l
    rows_per_w = n_rows // NW
    seqs_per_w = rows_per_w // NPOS
    n_chunks = rows_per_w // CHUNK
    sub_sizes = []
    off = 0
    while off < CHUNK:
        sub_sizes.append(min(SUB, CHUNK - off))
        off += min(SUB, CHUNK - off)
    mesh = plsc.VectorSubcoreMesh(core_axis_name="c", subcore_axis_name="s")

    def body(seq_hbm, lbl_hbm, tab_hbm, seg_hbm, pe_hbm, a2_hbm, b2_hbm,
             out_hbm, idx_s, idx_p, lblv, xbuf, ybuf, pebuf, segbuf,
             a2v_m, b2v_m, sem):
        wid = lax.axis_index("s") * NC + lax.axis_index("c")
        pltpu.sync_copy(pe_hbm.at[0, pl.ds(0, NPOS)], pebuf)
        pltpu.sync_copy(seg_hbm, segbuf)
        pltpu.sync_copy(a2_hbm, a2v_m)
        pltpu.sync_copy(b2_hbm, b2v_m)
        a2v = [a2v_m[pl.ds(c * 16, 16)] for c in range(EC)]
        b2v = [b2v_m[pl.ds(c * 16, 16)] for c in range(EC)]
        seg0 = [segbuf[0, pl.ds(c * 16, 16)] for c in range(EC)]
        dseg = [segbuf[1, pl.ds(c * 16, 16)] - seg0[c] for c in range(EC)]
        iota = jnp.arange(16, dtype=jnp.int32)
        bfly = [iota ^ m for m in (8, 4, 2, 1)]
        lanes = [jnp.full((16,), u, dtype=jnp.int32) for u in range(GROUP)]

        def hsum(v):
            for bidx in bfly:
                v = v + _perm(v, bidx)
            return v

        def process_row(r, svf, h):
            sq = jnp.where(r >= NPOS, 1, 0).astype(jnp.int32)
            l = r - sq * NPOS
            xp = []
            for c in range(EC):
                x = xbuf[r, pl.ds(h * E + c * 16, 16)]
                p = pebuf[l, pl.ds(c * 16, 16)]
                sg = svf * dseg[c] + seg0[c]
                xp.append((x + p) + sg)
            s1v = (xp[0] + xp[1]) + (xp[2] + xp[3])
            s2v = (xp[0] * xp[0] + xp[1] * xp[1]) + (xp[2] * xp[2] + xp[3] * xp[3])
            st = hsum(s1v)
            sst = hsum(s2v)
            mean = st * (1.0 / E)
            var = (sst - st * mean) * (1.0 / (E - 1))
            v = jnp.maximum(var, 1e-20)
            i = lax.bitcast_convert_type(v, jnp.int32)
            i = jnp.int32(0x5F3759DF) - (i >> 1)
            rsq = lax.bitcast_convert_type(i, jnp.float32)
            rsq = rsq * (1.5 - 0.5 * v * rsq * rsq)
            d = v * rsq + 1e-6
            inv = rsq * (2.0 - d * rsq)
            for c in range(EC):
                y = (xp[c] - mean) * inv * a2v[c] + b2v[c]
                ybuf[sq, l, pl.ds(c * 16, 16)] = y

        def chunk_body(ch, carry):
            base = wid * rows_per_w + ch * CHUNK
            b0 = wid * seqs_per_w + ch * SEQ_PER_CHUNK
            pltpu.sync_copy(seq_hbm.at[pl.ds(base, CHUNK)], idx_s)
            pltpu.sync_copy(lbl_hbm.at[pl.ds(base, CHUNK)], lblv)

            @plsc.parallel_loop(0, CHUNK // 16)
            def pidx_body(k):
                ids = idx_s[pl.ds(k * 16, 16)]
                idx_p[pl.ds(k * 16, 16)] = (
                    ((ids >> 13) << 12) | (ids & (RH - 1)))
            copies = []
            off = 0
            for sz in sub_sizes:
                copies.append(pltpu.async_copy(
                    tab_hbm.at[idx_p.at[pl.ds(off, sz)]],
                    xbuf.at[pl.ds(off, sz)], sem))
                off += sz
            for cp in copies:
                cp.wait()

            @plsc.parallel_loop(0, CHUNK // GROUP, 1, unroll=2)
            def row_body(g):
                lblf = lblv[pl.ds(g * GROUP, GROUP)].astype(jnp.float32)
                hvec = (idx_s[pl.ds(g * GROUP, GROUP)] >> 12) & 1
                for u in range(GROUP):
                    process_row(g * GROUP + u, _perm(lblf, lanes[u]), hvec[u])
            pltpu.sync_copy(ybuf, out_hbm.at[pl.ds(b0, SEQ_PER_CHUNK)])
            return carry

        lax.fori_loop(0, n_chunks, chunk_body, 0)

    call = pl.kernel(
        body,
        out_type=jax.ShapeDtypeStruct((n_b, n_l, E), jnp.float32),
        mesh=mesh,
        scratch_types=[
            pltpu.VMEM((CHUNK,), jnp.int32),
            pltpu.VMEM((CHUNK,), jnp.int32),
            pltpu.VMEM((CHUNK,), jnp.int32),
            pltpu.VMEM((CHUNK, 2 * E), jnp.float32),
            pltpu.VMEM((SEQ_PER_CHUNK, NPOS, E), jnp.float32),
            pltpu.VMEM((NPOS, E), jnp.float32),
            pltpu.VMEM((2, E), jnp.float32),
            pltpu.VMEM((E,), jnp.float32),
            pltpu.VMEM((E,), jnp.float32),
            pltpu.SemaphoreType.DMA,
        ],
        compiler_params=pltpu.CompilerParams(use_tc_tiling_on_sc=True),
    )
    return call(seq1d, lbl, tab2, seg_table, pe, a2, b2)


def kernel(sequence, segment_label, token_table, segment_table, pe, a_2, b_2):
    b, l = sequence.shape
    n_rows = b * l
    seq1d = sequence.reshape(n_rows)
    lbl = segment_label.reshape(n_rows)
    tab2 = _repack(token_table.T)
    return _sc_embed_ln(seq1d, lbl, tab2, segment_table, pe,
                        a_2, b_2, b, l)
